# baseline (speedup 1.0000x reference)
import functools

import jax
import jax.numpy as jnp
from jax import lax
from jax.experimental import pallas as pl
from jax.experimental.pallas import tpu as pltpu
from jax.experimental.pallas import tpu_sc as plsc

_VOCAB = 1000000
_D = 64
_H = 256
_B = 4096
_SEQ = 200

_NC = 2
_NS = 16
_NW = _NC * _NS
_BPW = _B // _NW

_SEQ_A = 104
_SEQ_B = _SEQ - _SEQ_A
_NV = _D // 16


def _pool_body(wi_hbm, tbl_hbm, out_hbm, idx_v, rows_a, rows_b, out_v, sem_a, sem_b):
  c = lax.axis_index("c")
  s = lax.axis_index("s")
  wid = s * _NC + c
  base = wid * _BPW

  pltpu.sync_copy(wi_hbm.at[pl.ds(base, _BPW)], idx_v)

  def reduce_rows(rows_ref, n):
    def jbody(j, acc):
      return tuple(
          acc[d] + rows_ref[j, pl.ds(d * 16, 16)] for d in range(_NV)
      )
    zero = tuple(jnp.zeros((16,), jnp.float32) for _ in range(_NV))
    return lax.fori_loop(0, n, jbody, zero)

  def row_body(r, carry):
    cp_a = pltpu.async_copy(
        tbl_hbm.at[idx_v.at[r, pl.ds(0, _SEQ_A)]], rows_a, sem_a)
    cp_b = pltpu.async_copy(
        tbl_hbm.at[idx_v.at[r, pl.ds(_SEQ_A, _SEQ_B)]], rows_b, sem_b)
    cp_a.wait()
    acc_a = reduce_rows(rows_a, _SEQ_A)
    cp_b.wait()
    acc_b = reduce_rows(rows_b, _SEQ_B)
    for d in range(_NV):
      out_v[r, pl.ds(d * 16, 16)] = acc_a[d] + acc_b[d]
    return carry

  lax.fori_loop(0, _BPW, row_body, 0)
  pltpu.sync_copy(out_v, out_hbm.at[pl.ds(base, _BPW)])


@jax.jit
def _pool(word_indices, table):
  mesh = plsc.VectorSubcoreMesh(
      core_axis_name="c", subcore_axis_name="s",
      num_cores=_NC, num_subcores=_NS)
  return pl.kernel(
      _pool_body,
      out_type=jax.ShapeDtypeStruct((_B, _D), jnp.float32),
      mesh=mesh,
      compiler_params=pltpu.CompilerParams(use_tc_tiling_on_sc=False),
      scratch_types=[
          pltpu.VMEM((_BPW, _SEQ), jnp.int32),
          pltpu.VMEM((_SEQ_A, _D), jnp.float32),
          pltpu.VMEM((_SEQ_B, _D), jnp.float32),
          pltpu.VMEM((_BPW, _D), jnp.float32),
          pltpu.SemaphoreType.DMA,
          pltpu.SemaphoreType.DMA,
      ],
  )(word_indices, table)


def _mlp_kernel(x_ref, w1_ref, b1_ref, w2_ref, b2_ref, o_ref):
  x = x_ref[...] * (1.0 / _SEQ)
  h = jnp.dot(x, w1_ref[...], preferred_element_type=jnp.float32)
  h = jnp.maximum(h + b1_ref[...], 0.0)
  logits = jnp.dot(h, w2_ref[...], preferred_element_type=jnp.float32)
  logits = logits + b2_ref[...]
  m = jnp.max(logits, axis=1, keepdims=True)
  e = jnp.exp(logits - m)
  o_ref[...] = e / jnp.sum(e, axis=1, keepdims=True)


@jax.jit
def _mlp(sums, W1, b1, W2p, b2p):
  return pl.pallas_call(
      _mlp_kernel,
      out_shape=jax.ShapeDtypeStruct((_B, 128), jnp.float32),
  )(sums, W1, b1, W2p, b2p)


def kernel(word_indices, table, W1, b1, W2, b2):
  sums = _pool(word_indices, table)
  W2p = jnp.pad(W2, ((0, 0), (0, 128 - W2.shape[1])))
  b2p = jnp.concatenate(
      [b2, jnp.full((128 - b2.shape[0],), -1e30, jnp.float32)])
  out = _mlp(sums, W1, b1.reshape(1, _H), W2p, b2p.reshape(1, 128))
  return out[:, :2]


# speedup vs baseline: 1.1577x; 1.1577x over previous
import functools

import jax
import jax.numpy as jnp
from jax import lax
from jax.experimental import pallas as pl
from jax.experimental.pallas import tpu as pltpu
from jax.experimental.pallas import tpu_sc as plsc

_VOCAB = 1000000
_D = 64
_H = 256
_B = 4096
_SEQ = 200

_NC = 2
_NS = 16
_NW = _NC * _NS
_BPW = _B // _NW

_SEQ_A = 104
_SEQ_B = _SEQ - _SEQ_A
_NV = _D // 16


_UNROLL = 8


def _pool_body(wi_hbm, tbl_hbm, out_hbm, idx_v, rows_a0, rows_b0, rows_a1,
               rows_b1, out_v, sem_a0, sem_b0, sem_a1, sem_b1):
  c = lax.axis_index("c")
  s = lax.axis_index("s")
  wid = s * _NC + c
  base = wid * _BPW

  pltpu.sync_copy(wi_hbm.at[pl.ds(base, _BPW)], idx_v)

  def issue(r, rows_a, rows_b, sem_a, sem_b):
    pltpu.async_copy(
        tbl_hbm.at[idx_v.at[r, pl.ds(0, _SEQ_A)]], rows_a, sem_a)
    pltpu.async_copy(
        tbl_hbm.at[idx_v.at[r, pl.ds(_SEQ_A, _SEQ_B)]], rows_b, sem_b)

  def wait(rows_a, rows_b, sem_a, sem_b):
    pltpu.make_async_copy(tbl_hbm.at[pl.ds(0, _SEQ_A)], rows_a, sem_a).wait()
    pltpu.make_async_copy(tbl_hbm.at[pl.ds(0, _SEQ_B)], rows_b, sem_b).wait()

  def reduce_rows(rows_ref, n, acc):
    def jbody(j, acc):
      for k in range(_UNROLL):
        acc = tuple(
            acc[d] + rows_ref[j * _UNROLL + k, pl.ds(d * 16, 16)]
            for d in range(_NV))
      return acc
    return lax.fori_loop(0, n // _UNROLL, jbody, acc)

  def consume(r, rows_a, rows_b, sem_a, sem_b):
    wait(rows_a, rows_b, sem_a, sem_b)
    zero = tuple(jnp.zeros((16,), jnp.float32) for _ in range(_NV))
    acc = reduce_rows(rows_a, _SEQ_A, zero)
    acc = reduce_rows(rows_b, _SEQ_B, acc)
    for d in range(_NV):
      out_v[r, pl.ds(d * 16, 16)] = acc[d]

  issue(0, rows_a0, rows_b0, sem_a0, sem_b0)

  def pair_body(p, carry):
    r0 = 2 * p
    issue(r0 + 1, rows_a1, rows_b1, sem_a1, sem_b1)
    consume(r0, rows_a0, rows_b0, sem_a0, sem_b0)
    issue(jnp.minimum(r0 + 2, _BPW - 1), rows_a0, rows_b0, sem_a0, sem_b0)
    consume(r0 + 1, rows_a1, rows_b1, sem_a1, sem_b1)
    return carry

  lax.fori_loop(0, _BPW // 2, pair_body, 0)
  wait(rows_a0, rows_b0, sem_a0, sem_b0)
  pltpu.sync_copy(out_v, out_hbm.at[pl.ds(base, _BPW)])


@jax.jit
def _pool(word_indices, table):
  mesh = plsc.VectorSubcoreMesh(
      core_axis_name="c", subcore_axis_name="s",
      num_cores=_NC, num_subcores=_NS)
  return pl.kernel(
      _pool_body,
      out_type=jax.ShapeDtypeStruct((_B, _D), jnp.float32),
      mesh=mesh,
      compiler_params=pltpu.CompilerParams(use_tc_tiling_on_sc=False),
      scratch_types=[
          pltpu.VMEM((_BPW, _SEQ), jnp.int32),
          pltpu.VMEM((_SEQ_A, _D), jnp.float32),
          pltpu.VMEM((_SEQ_B, _D), jnp.float32),
          pltpu.VMEM((_SEQ_A, _D), jnp.float32),
          pltpu.VMEM((_SEQ_B, _D), jnp.float32),
          pltpu.VMEM((_BPW, _D), jnp.float32),
          pltpu.SemaphoreType.DMA,
          pltpu.SemaphoreType.DMA,
          pltpu.SemaphoreType.DMA,
          pltpu.SemaphoreType.DMA,
      ],
  )(word_indices, table)


def _mlp_kernel(x_ref, w1_ref, b1_ref, w2_ref, b2_ref, o_ref):
  x = x_ref[...] * (1.0 / _SEQ)
  h = jnp.dot(x, w1_ref[...], preferred_element_type=jnp.float32)
  h = jnp.maximum(h + b1_ref[...], 0.0)
  logits = jnp.dot(h, w2_ref[...], preferred_element_type=jnp.float32)
  logits = logits + b2_ref[...]
  m = jnp.max(logits, axis=1, keepdims=True)
  e = jnp.exp(logits - m)
  o_ref[...] = e / jnp.sum(e, axis=1, keepdims=True)


@jax.jit
def _mlp(sums, W1, b1, W2p, b2p):
  return pl.pallas_call(
      _mlp_kernel,
      out_shape=jax.ShapeDtypeStruct((_B, 128), jnp.float32),
  )(sums, W1, b1, W2p, b2p)


def kernel(word_indices, table, W1, b1, W2, b2):
  sums = _pool(word_indices, table)
  W2p = jnp.pad(W2, ((0, 0), (0, 128 - W2.shape[1])))
  b2p = jnp.concatenate(
      [b2, jnp.full((128 - b2.shape[0],), -1e30, jnp.float32)])
  out = _mlp(sums, W1, b1.reshape(1, _H), W2p, b2p.reshape(1, 128))
  return out[:, :2]
